# Initial kernel scaffold; baseline (speedup 1.0000x reference)
import jax
import jax.numpy as jnp
from jax.experimental import pallas as pl


def kernel(x, edge_attr, params, edge_index, batch):
    raise NotImplementedError("write your pallas kernel here")


# speedup vs baseline: 1.7607x; 1.7607x over previous
import functools

import jax
import jax.numpy as jnp
from jax import lax
from jax.experimental import pallas as pl
from jax.experimental.pallas import tpu as pltpu
from jax.experimental.pallas import tpu_sc as plsc

N = 50000
E = 800000
H = 128
G = 256
NC = 2
NS = 16
NW = NC * NS

NB = 2000
EB = 4000
EG = E // 128
FCS = 8
FCC = 8
RPT = N // NS


def _sp(x):
    return jnp.logaddexp(x, 0.0)


def _emb_body(x_ref, we_ref, be_ref, wi_ref, wj_ref, h_ref, ha_ref, hb_ref):
    h = jnp.dot(x_ref[...], we_ref[...], preferred_element_type=jnp.float32, precision=lax.Precision.HIGHEST)
    h = h + be_ref[...]
    h_ref[...] = h
    ha_ref[...] = jnp.dot(h, wi_ref[...], preferred_element_type=jnp.float32, precision=lax.Precision.HIGHEST)
    hb_ref[...] = jnp.dot(h, wj_ref[...], preferred_element_type=jnp.float32, precision=lax.Precision.HIGHEST)


def _emb_call(x, we, be, wi, wj):
    grid = (N // NB,)
    return pl.pallas_call(
        _emb_body,
        grid=grid,
        in_specs=[
            pl.BlockSpec((NB, 92), lambda i: (i, 0)),
            pl.BlockSpec((92, H), lambda i: (0, 0)),
            pl.BlockSpec((1, H), lambda i: (0, 0)),
            pl.BlockSpec((H, H), lambda i: (0, 0)),
            pl.BlockSpec((H, H), lambda i: (0, 0)),
        ],
        out_specs=[
            pl.BlockSpec((NB, H), lambda i: (i, 0)),
            pl.BlockSpec((NB, H), lambda i: (i, 0)),
            pl.BlockSpec((NB, H), lambda i: (i, 0)),
        ],
        out_shape=[jax.ShapeDtypeStruct((N, H), jnp.float32)] * 3,
    )(x, we, be, wi, wj)


def _prep_body(we2_ref, wf_ref, bn1_ref, be2_ref, w2f_ref, cb_ref):
    w2f_ref[...] = jnp.dot(we2_ref[...], wf_ref[...],
                           preferred_element_type=jnp.float32, precision=lax.Precision.HIGHEST)
    cb_ref[...] = bn1_ref[...] + jnp.dot(be2_ref[...], wf_ref[...],
                                         preferred_element_type=jnp.float32, precision=lax.Precision.HIGHEST)


def _prep_call(we2, wf, bn1, be2):
    return pl.pallas_call(
        _prep_body,
        out_shape=[jax.ShapeDtypeStruct((H, H), jnp.float32),
                   jax.ShapeDtypeStruct((1, H), jnp.float32)],
    )(we2, wf, bn1, be2)


def _edge_body(ea_ref, g_ref, w1_ref, b1_ref, w2_ref, cb_ref, s_ref):
    u = _sp(jnp.dot(ea_ref[...], w1_ref[...],
                    preferred_element_type=jnp.float32, precision=lax.Precision.HIGHEST) + b1_ref[...])
    t = g_ref[...] + jnp.dot(u, w2_ref[...],
                             preferred_element_type=jnp.float32, precision=lax.Precision.HIGHEST) + cb_ref[...]
    s_ref[...] = _sp(t)


def _edge_call(ea, g, w1, b1, w2, cb):
    grid = (E // EB,)
    return pl.pallas_call(
        _edge_body,
        grid=grid,
        in_specs=[
            pl.BlockSpec((EB, 41), lambda i: (i, 0)),
            pl.BlockSpec((EB, H), lambda i: (i, 0)),
            pl.BlockSpec((41, H), lambda i: (0, 0)),
            pl.BlockSpec((1, H), lambda i: (0, 0)),
            pl.BlockSpec((H, H), lambda i: (0, 0)),
            pl.BlockSpec((1, H), lambda i: (0, 0)),
        ],
        out_specs=pl.BlockSpec((EB, H), lambda i: (i, 0)),
        out_shape=jax.ShapeDtypeStruct((E, H), jnp.float32),
    )(ea, g, w1, b1, w2, cb)


def _c1_body(h_ref, agg_ref, cnt_ref, w2_ref, b2_ref, hn_ref, st_ref):
    i = pl.program_id(0)
    indeg = cnt_ref[0, :, 0] + cnt_ref[1, :, 0]
    hn = h_ref[...] + jnp.dot(agg_ref[...], w2_ref[...],
                              preferred_element_type=jnp.float32, precision=lax.Precision.HIGHEST)
    hn = hn + indeg[:, None] * b2_ref[...]
    hn_ref[...] = hn

    @pl.when(i == 0)
    def _():
        st_ref[...] = jnp.zeros_like(st_ref)

    st_ref[0:1, :] += jnp.sum(hn, axis=0, keepdims=True)
    st_ref[1:2, :] += jnp.sum(hn * hn, axis=0, keepdims=True)


def _c1_call(h, agg, cnt, w2, b2):
    grid = (N // NB,)
    return pl.pallas_call(
        _c1_body,
        grid=grid,
        in_specs=[
            pl.BlockSpec((NB, H), lambda i: (i, 0)),
            pl.BlockSpec((NB, H), lambda i: (i, 0)),
            pl.BlockSpec((2, NB, FCC), lambda i: (0, i, 0)),
            pl.BlockSpec((H, H), lambda i: (0, 0)),
            pl.BlockSpec((1, H), lambda i: (0, 0)),
        ],
        out_specs=[
            pl.BlockSpec((NB, H), lambda i: (i, 0)),
            pl.BlockSpec((8, H), lambda i: (0, 0)),
        ],
        out_shape=[jax.ShapeDtypeStruct((N, H), jnp.float32),
                   jax.ShapeDtypeStruct((8, H), jnp.float32)],
    )(h, agg, cnt, w2, b2)


def _c2_body(hn_ref, st_ref, gam_ref, bet_ref, wi_ref, wj_ref,
             y_ref, ha_ref, hb_ref):
    mean = st_ref[0:1, :] * (1.0 / N)
    var = st_ref[1:2, :] * (1.0 / N) - mean * mean
    inv = lax.rsqrt(var + 1e-5)
    y = _sp((hn_ref[...] - mean) * inv * gam_ref[...] + bet_ref[...])
    y_ref[...] = y
    ha_ref[...] = jnp.dot(y, wi_ref[...], preferred_element_type=jnp.float32, precision=lax.Precision.HIGHEST)
    hb_ref[...] = jnp.dot(y, wj_ref[...], preferred_element_type=jnp.float32, precision=lax.Precision.HIGHEST)


def _c2_call(hn, st, gam, bet, wi, wj):
    grid = (N // NB,)
    return pl.pallas_call(
        _c2_body,
        grid=grid,
        in_specs=[
            pl.BlockSpec((NB, H), lambda i: (i, 0)),
            pl.BlockSpec((8, H), lambda i: (0, 0)),
            pl.BlockSpec((1, H), lambda i: (0, 0)),
            pl.BlockSpec((1, H), lambda i: (0, 0)),
            pl.BlockSpec((H, H), lambda i: (0, 0)),
            pl.BlockSpec((H, H), lambda i: (0, 0)),
        ],
        out_specs=[
            pl.BlockSpec((NB, H), lambda i: (i, 0)),
            pl.BlockSpec((NB, H), lambda i: (i, 0)),
            pl.BlockSpec((NB, H), lambda i: (i, 0)),
        ],
        out_shape=[jax.ShapeDtypeStruct((N, H), jnp.float32)] * 3,
    )(hn, st, gam, bet, wi, wj)


def _c2last_body(hn_ref, st_ref, gam_ref, bet_ref, y_ref):
    mean = st_ref[0:1, :] * (1.0 / N)
    var = st_ref[1:2, :] * (1.0 / N) - mean * mean
    inv = lax.rsqrt(var + 1e-5)
    y_ref[...] = _sp((hn_ref[...] - mean) * inv * gam_ref[...] + bet_ref[...])


def _c2last_call(hn, st, gam, bet):
    grid = (N // NB,)
    return pl.pallas_call(
        _c2last_body,
        grid=grid,
        in_specs=[
            pl.BlockSpec((NB, H), lambda i: (i, 0)),
            pl.BlockSpec((8, H), lambda i: (0, 0)),
            pl.BlockSpec((1, H), lambda i: (0, 0)),
            pl.BlockSpec((1, H), lambda i: (0, 0)),
        ],
        out_specs=pl.BlockSpec((NB, H), lambda i: (i, 0)),
        out_shape=jax.ShapeDtypeStruct((N, H), jnp.float32),
    )(hn, st, gam, bet)


def _pool_body(y_ref, b_ref, sums_ref, cnts_ref):
    i = pl.program_id(0)
    b = b_ref[0, 0, :]
    onehot = (b[:, None] == lax.broadcasted_iota(jnp.int32, (1, G), 1))
    onehot = onehot.astype(jnp.float32)
    ps = lax.dot_general(onehot, y_ref[...], (((0,), (0,)), ((), ())),
                         preferred_element_type=jnp.float32, precision=lax.Precision.HIGHEST)
    pc = lax.dot_general(onehot, jnp.ones_like(y_ref[...]),
                         (((0,), (0,)), ((), ())),
                         preferred_element_type=jnp.float32, precision=lax.Precision.HIGHEST)

    @pl.when(i == 0)
    def _():
        sums_ref[...] = jnp.zeros_like(sums_ref)
        cnts_ref[...] = jnp.zeros_like(cnts_ref)

    sums_ref[...] += ps
    cnts_ref[...] += pc


def _pool_call(y, batch3):
    grid = (N // NB,)
    return pl.pallas_call(
        _pool_body,
        grid=grid,
        in_specs=[
            pl.BlockSpec((NB, H), lambda i: (i, 0)),
            pl.BlockSpec((1, 1, NB), lambda i: (i, 0, 0)),
        ],
        out_specs=[
            pl.BlockSpec((G, H), lambda i: (0, 0)),
            pl.BlockSpec((G, H), lambda i: (0, 0)),
        ],
        out_shape=[jax.ShapeDtypeStruct((G, H), jnp.float32),
                   jax.ShapeDtypeStruct((G, H), jnp.float32)],
    )(y, batch3)


def _head_body(sums_ref, cnts_ref, w1_ref, b1_ref, w2_ref, b2_ref,
               wo_ref, bo_ref, out_ref):
    cnt = jnp.maximum(cnts_ref[...], 1.0)
    mean = sums_ref[...] / cnt
    w1 = w1_ref[0:H, :] + w1_ref[H:2 * H, :]
    z = _sp(jnp.dot(mean, w1, preferred_element_type=jnp.float32, precision=lax.Precision.HIGHEST) + b1_ref[...])
    z = _sp(jnp.dot(z, w2_ref[...], preferred_element_type=jnp.float32, precision=lax.Precision.HIGHEST)
            + b2_ref[...])
    out_ref[...] = jnp.dot(z, wo_ref[...],
                           preferred_element_type=jnp.float32, precision=lax.Precision.HIGHEST) + bo_ref[...]


def _head_call(sums, cnts, w1, b1, w2, b2, wo, bo):
    return pl.pallas_call(
        _head_body,
        out_shape=jax.ShapeDtypeStruct((G, 8), jnp.float32),
    )(sums, cnts, w1, b1, w2, b2, wo, bo)


_MESH = plsc.VectorSubcoreMesh(core_axis_name="c", subcore_axis_name="s",
                               num_cores=NC, num_subcores=NS)

_GATHER_TRIPS = (EG + NW - 1) // NW


@functools.partial(
    pl.kernel,
    out_type=jax.ShapeDtypeStruct((E, H), jnp.float32),
    mesh=_MESH,
    compiler_params=pltpu.CompilerParams(use_tc_tiling_on_sc=False),
    scratch_types=[
        pltpu.VMEM((1, 128), jnp.int32),
        pltpu.VMEM((1, 128), jnp.int32),
        pltpu.VMEM((128, H), jnp.float32),
        pltpu.SemaphoreType.DMA,
    ],
)
def _sc_gather(ha_hbm, hb_hbm, dst2_hbm, src2_hbm, g_hbm,
               idxd_v, idxs_v, rows_v, sem):
    wid = lax.axis_index("s") * NC + lax.axis_index("c")

    @pl.loop(0, _GATHER_TRIPS)
    def _(k):
        gidx = k * NW + wid

        @pl.when(gidx < EG)
        def _():
            pltpu.sync_copy(dst2_hbm.at[pl.ds(gidx, 1)], idxd_v)
            pltpu.sync_copy(src2_hbm.at[pl.ds(gidx, 1)], idxs_v)
            pltpu.async_copy(ha_hbm.at[idxd_v.at[0]], rows_v, sem).wait()
            pltpu.async_copy(hb_hbm.at[idxs_v.at[0]], rows_v, sem,
                             add=True).wait()
            pltpu.sync_copy(rows_v, g_hbm.at[pl.ds(gidx * 128, 128)])


_SCAT_TRIPS = (EG + NS - 1) // NS


@functools.partial(
    pl.kernel,
    out_type=jax.ShapeDtypeStruct((N, H), jnp.float32),
    mesh=_MESH,
    compiler_params=pltpu.CompilerParams(use_tc_tiling_on_sc=False),
    scratch_types=[
        pltpu.VMEM((1, 128), jnp.int32),
        pltpu.VMEM((128, FCS), jnp.float32),
        pltpu.VMEM((625, FCS), jnp.float32),
        pltpu.VMEM_SHARED((N, FCS), jnp.float32),
        pltpu.SemaphoreType.DMA,
    ],
)
def _sc_scatter(s_hbm, dst2_hbm, zero_hbm, agg_hbm,
                idx_v, vals_v, zero_v, table_sh, sem):
    cid = lax.axis_index("c")
    tid = lax.axis_index("s")
    pltpu.sync_copy(zero_hbm, zero_v)
    for cc in range(8):
        c0 = (cid * 8 + cc) * FCS
        for z in range(RPT // 625):
            pltpu.sync_copy(zero_v,
                            table_sh.at[pl.ds(tid * RPT + z * 625, 625)])
        plsc.subcore_barrier()

        @pl.loop(0, _SCAT_TRIPS)
        def _(k):
            gidx = k * NS + tid

            @pl.when(gidx < EG)
            def _():
                pltpu.sync_copy(dst2_hbm.at[pl.ds(gidx, 1)], idx_v)
                pltpu.sync_copy(s_hbm.at[pl.ds(gidx * 128, 128),
                                         pl.ds(c0, FCS)], vals_v)
                pltpu.sync_copy(vals_v, table_sh.at[idx_v.at[0]], add=True)

        plsc.subcore_barrier()
        r0 = tid * RPT
        pltpu.sync_copy(table_sh.at[pl.ds(r0, RPT)],
                        agg_hbm.at[pl.ds(r0, RPT), pl.ds(c0, FCS)])
        plsc.subcore_barrier()


_CNT_TRIPS = (EG // NC + NS - 1) // NS


@functools.partial(
    pl.kernel,
    out_type=jax.ShapeDtypeStruct((2, N, FCC), jnp.float32),
    mesh=_MESH,
    compiler_params=pltpu.CompilerParams(use_tc_tiling_on_sc=False),
    scratch_types=[
        pltpu.VMEM((1, 128), jnp.int32),
        pltpu.VMEM((128, FCC), jnp.float32),
        pltpu.VMEM((625, FCC), jnp.float32),
        pltpu.VMEM_SHARED((N, FCC), jnp.float32),
        pltpu.SemaphoreType.DMA,
    ],
)
def _sc_count(dst2_hbm, ones_hbm, zero_hbm, cnt_hbm,
              idx_v, ones_v, zero_v, table_sh, sem):
    cid = lax.axis_index("c")
    tid = lax.axis_index("s")
    pltpu.sync_copy(zero_hbm, zero_v)
    pltpu.sync_copy(ones_hbm, ones_v)
    for z in range(RPT // 625):
        pltpu.sync_copy(zero_v, table_sh.at[pl.ds(tid * RPT + z * 625, 625)])
    plsc.subcore_barrier()
    half = EG // NC

    @pl.loop(0, _CNT_TRIPS)
    def _(k):
        gidx = cid * half + k * NS + tid

        @pl.when(k * NS + tid < half)
        def _():
            pltpu.sync_copy(dst2_hbm.at[pl.ds(gidx, 1)], idx_v)
            pltpu.sync_copy(ones_v, table_sh.at[idx_v.at[0]], add=True)

    plsc.subcore_barrier()
    r0 = tid * RPT
    pltpu.sync_copy(table_sh.at[pl.ds(r0, RPT)],
                    cnt_hbm.at[cid, pl.ds(r0, RPT)])


def kernel(x, edge_attr, params, edge_index, batch):
    src = edge_index[0]
    dst = edge_index[1]
    dst2 = dst.reshape(EG, 128)
    src2 = src.reshape(EG, 128)
    zeros_s = jnp.zeros((625, FCS), jnp.float32)
    zeros_c = jnp.zeros((625, FCC), jnp.float32)
    ones128 = jnp.ones((128, FCC), jnp.float32)
    batch3 = batch.reshape(N // NB, 1, NB)

    emb = params["emb"]
    convs = params["convs"]
    bns = params["bns"]

    def row(v):
        return v.reshape(1, -1)

    wi = [c["n1"]["W"][0:H, :] for c in convs]
    wj = [c["n1"]["W"][H:2 * H, :] for c in convs]
    wf = [c["n1"]["W"][2 * H:, :] for c in convs]

    cnt = _sc_count(dst2, ones128, zeros_c)

    h, ha, hb = _emb_call(x, emb["W"], row(emb["b"]), wi[0], wj[0])

    for l in range(3):
        c = convs[l]
        w2f, cb = _prep_call(c["e2"]["W"], wf[l], row(c["n1"]["b"]),
                             row(c["e2"]["b"]))
        g = _sc_gather(ha, hb, dst2, src2)
        s = _edge_call(edge_attr, g, c["e1"]["W"], row(c["e1"]["b"]), w2f, cb)
        agg = _sc_scatter(s, dst2, zeros_s)
        hn, st = _c1_call(h, agg, cnt, c["n2"]["W"],
                          row(c["n2"]["b"]))
        bn = bns[l]
        if l < 2:
            h, ha, hb = _c2_call(hn, st, row(bn["gamma"]), row(bn["beta"]),
                                 wi[l + 1], wj[l + 1])
        else:
            y = _c2last_call(hn, st, row(bn["gamma"]), row(bn["beta"]))

    sums, cnts = _pool_call(y, batch3)
    fc0, fc1 = params["fcs"]
    wo = jnp.pad(params["out"]["W"], ((0, 0), (0, 7)))
    bo = jnp.pad(row(params["out"]["b"]), ((0, 0), (0, 7)))
    out8 = _head_call(sums, cnts, fc0["W"], row(fc0["b"]),
                      fc1["W"], row(fc1["b"]), wo, bo)
    return out8[:, 0:1]
